# Initial kernel scaffold; baseline (speedup 1.0000x reference)
import jax
import jax.numpy as jnp
from jax.experimental import pallas as pl


def kernel(x, tok_table, pos_table, gamma, beta):
    raise NotImplementedError("write your pallas kernel here")


# speedup vs baseline: 3.4858x; 3.4858x over previous
import functools

import jax
import jax.numpy as jnp
from jax.experimental import pallas as pl
from jax.experimental.pallas import tpu as pltpu

_VOCAB = 6
_EPS = 1e-5


def _body(x_ref, tok_ref, pos_ref, gamma_ref, beta_ref, out_ref):
    pos = pos_ref[...]
    gamma = gamma_ref[...]
    beta = beta_ref[...]
    xb = x_ref[...]

    b_blk = xb.shape[0]
    s, d = pos.shape
    acc = None
    for v in range(_VOCAB):
        row = tok_ref[pl.ds(v, 1), :] + pos
        mean = jnp.mean(row, axis=-1, keepdims=True)
        cent = row - mean
        var = jnp.mean(cent * cent, axis=-1, keepdims=True)
        normed = cent * jax.lax.rsqrt(var + _EPS)
        normed = normed * gamma + beta
        if acc is None:
            acc = jnp.broadcast_to(normed[None], (b_blk, s, d))
        else:
            mask = xb == v
            acc = jnp.where(mask, normed[None], acc)
    out_ref[...] = acc


@functools.partial(jax.jit, static_argnames=("b_blk",))
def _run(x, tok_table, pos_table, gamma, beta, b_blk=64):
    batch, seq = x.shape
    d = tok_table.shape[-1]
    grid = batch // b_blk
    return pl.pallas_call(
        _body,
        grid=(grid,),
        in_specs=[
            pl.BlockSpec((b_blk, seq, 1), lambda i: (i, 0, 0)),
            pl.BlockSpec((_VOCAB, d), lambda i: (0, 0)),
            pl.BlockSpec((seq, d), lambda i: (0, 0)),
            pl.BlockSpec((1, d), lambda i: (0, 0)),
            pl.BlockSpec((1, d), lambda i: (0, 0)),
        ],
        out_specs=pl.BlockSpec((b_blk, seq, d), lambda i: (i, 0, 0)),
        out_shape=jax.ShapeDtypeStruct((batch, seq, d), jnp.float32),
        compiler_params=pltpu.CompilerParams(
            dimension_semantics=("arbitrary",),
        ),
    )(x[:, :, None], tok_table, pos_table, gamma.reshape(1, d), beta.reshape(1, d))


def kernel(x, tok_table, pos_table, gamma, beta):
    return _run(x, tok_table, pos_table, gamma, beta)


# speedup vs baseline: 3.6032x; 1.0337x over previous
import functools

import jax
import jax.numpy as jnp
from jax.experimental import pallas as pl
from jax.experimental.pallas import tpu as pltpu

_VOCAB = 6
_EPS = 1e-5


def _body(x_ref, tok_ref, pos_ref, gamma_ref, beta_ref, out_ref, t_ref):
    seq, d = pos_ref.shape
    b_blk = x_ref.shape[0]

    @pl.when(pl.program_id(0) == 0)
    def _build_table():
        pos = pos_ref[...]
        gamma = gamma_ref[...]
        beta = beta_ref[...]
        for v in range(_VOCAB):
            row = tok_ref[pl.ds(v, 1), :] + pos
            mean = jnp.mean(row, axis=-1, keepdims=True)
            cent = row - mean
            var = jnp.mean(cent * cent, axis=-1, keepdims=True)
            normed = cent * jax.lax.rsqrt(var + _EPS)
            normed = normed * gamma + beta
            t_ref[v, :, :] = normed.astype(jnp.bfloat16)

    xb = x_ref[...]
    iota_v = jax.lax.broadcasted_iota(jnp.int32, (b_blk, _VOCAB), 1)
    for s in range(seq):
        oh = (xb[:, s : s + 1] == iota_v).astype(jnp.bfloat16)
        w = t_ref[:, s, :]
        res = jax.lax.dot_general(
            oh, w, (((1,), (0,)), ((), ())),
            preferred_element_type=jnp.float32,
        )
        out_ref[:, s, :] = res


@functools.partial(jax.jit, static_argnames=("b_blk",))
def _run(x, tok_table, pos_table, gamma, beta, b_blk=64):
    batch, seq = x.shape
    d = tok_table.shape[-1]
    grid = batch // b_blk
    return pl.pallas_call(
        _body,
        grid=(grid,),
        in_specs=[
            pl.BlockSpec((b_blk, seq), lambda i: (i, 0)),
            pl.BlockSpec((_VOCAB, d), lambda i: (0, 0)),
            pl.BlockSpec((seq, d), lambda i: (0, 0)),
            pl.BlockSpec((1, d), lambda i: (0, 0)),
            pl.BlockSpec((1, d), lambda i: (0, 0)),
        ],
        out_specs=pl.BlockSpec((b_blk, seq, d), lambda i: (i, 0, 0)),
        out_shape=jax.ShapeDtypeStruct((batch, seq, d), jnp.float32),
        scratch_shapes=[pltpu.VMEM((_VOCAB, seq, d), jnp.bfloat16)],
        compiler_params=pltpu.CompilerParams(
            dimension_semantics=("arbitrary",),
        ),
    )(x, tok_table, pos_table, gamma.reshape(1, d), beta.reshape(1, d))


def kernel(x, tok_table, pos_table, gamma, beta):
    return _run(x, tok_table, pos_table, gamma, beta)


# speedup vs baseline: 4.2690x; 1.1848x over previous
import functools

import jax
import jax.numpy as jnp
from jax.experimental import pallas as pl
from jax.experimental.pallas import tpu as pltpu

_VOCAB = 6
_EPS = 1e-5


def _body(xt_ref, tok_ref, pos_ref, gamma_ref, beta_ref, out_ref, t_ref):
    seq, d = pos_ref.shape
    b_blk = xt_ref.shape[2]

    @pl.when(pl.program_id(0) == 0)
    def _build_table():
        pos = pos_ref[...]
        gamma = gamma_ref[...]
        beta = beta_ref[...]
        for v in range(_VOCAB):
            row = tok_ref[pl.ds(v, 1), :] + pos
            mean = jnp.mean(row, axis=-1, keepdims=True)
            cent = row - mean
            var = jnp.mean(cent * cent, axis=-1, keepdims=True)
            normed = cent * jax.lax.rsqrt(var + _EPS)
            normed = normed * gamma + beta
            t_ref[v, :, :] = normed.astype(jnp.bfloat16)

    for b in range(b_blk):
        xcol = xt_ref[0, :, b : b + 1].astype(jnp.bfloat16)
        xb = jnp.broadcast_to(xcol, (seq, d))
        acc = t_ref[0, :, :]
        for v in range(1, _VOCAB):
            acc = jnp.where(xb >= v, t_ref[v, :, :], acc)
        out_ref[b, :, :] = acc.astype(jnp.float32)


@functools.partial(jax.jit, static_argnames=("b_blk",))
def _run(x, tok_table, pos_table, gamma, beta, b_blk=32):
    batch, seq = x.shape
    d = tok_table.shape[-1]
    grid = batch // b_blk
    call = pl.pallas_call(
        _body,
        grid=(grid,),
        in_specs=[
            pl.BlockSpec((1, seq, b_blk), lambda i: (i, 0, 0)),
            pl.BlockSpec((_VOCAB, d), lambda i: (0, 0)),
            pl.BlockSpec((seq, d), lambda i: (0, 0)),
            pl.BlockSpec((1, d), lambda i: (0, 0)),
            pl.BlockSpec((1, d), lambda i: (0, 0)),
        ],
        out_specs=pl.BlockSpec((b_blk, seq, d), lambda i: (i, 0, 0)),
        out_shape=jax.ShapeDtypeStruct((batch, seq, d), jnp.float32),
        scratch_shapes=[pltpu.VMEM((_VOCAB, seq, d), jnp.bfloat16)],
        compiler_params=pltpu.CompilerParams(
            dimension_semantics=("arbitrary",),
        ),
    )
    xt = x.reshape(grid, b_blk, seq).transpose(0, 2, 1)
    return call(xt, tok_table, pos_table, gamma.reshape(1, d), beta.reshape(1, d))


def kernel(x, tok_table, pos_table, gamma, beta):
    return _run(x, tok_table, pos_table, gamma, beta)
